# baseline (speedup 1.0000x reference)
import functools

import jax
import jax.numpy as jnp
from jax.experimental import pallas as pl
from jax.experimental.pallas import tpu as pltpu

_R = 512


def _body(mid_ref, xc_ref, xp_ref, tok_ref, o_ref):
    j = pl.program_id(1)
    cur = xc_ref[0]
    prev8 = xp_ref[0]
    shifted = jnp.concatenate([prev8[7:8], cur[: _R - 1]], axis=0)

    mid = mid_ref[0]
    tok = tok_ref[...]
    tok_ids = jax.lax.broadcasted_iota(jnp.int32, tok.shape, 0)
    tokrow = jnp.sum(jnp.where(tok_ids == mid, tok, 0.0), axis=0)

    row_ids = jax.lax.broadcasted_iota(jnp.int32, shifted.shape, 0)
    take_tok = jnp.logical_and(row_ids == 0, j == 0)
    o_ref[0] = jnp.where(take_tok, tokrow[None, :], shifted)


def kernel(x, modality_id, tokens):
    B, S, D = x.shape
    mid = jnp.reshape(jnp.asarray(modality_id, jnp.int32), (1,))
    nj = S // _R + 1
    grid = (B, nj)
    call = pl.pallas_call(
        functools.partial(_body),
        grid_spec=pltpu.PrefetchScalarGridSpec(
            num_scalar_prefetch=1,
            grid=grid,
            in_specs=[
                pl.BlockSpec(
                    (1, _R, D),
                    lambda b, j, *_: (b, jnp.minimum(j, S // _R - 1), 0),
                ),
                pl.BlockSpec(
                    (1, 8, D),
                    lambda b, j, *_: (b, jnp.maximum(j * (_R // 8) - 1, 0), 0),
                ),
                pl.BlockSpec((8, D), lambda b, j, *_: (0, 0)),
            ],
            out_specs=pl.BlockSpec((1, _R, D), lambda b, j, *_: (b, j, 0)),
        ),
        out_shape=jax.ShapeDtypeStruct((B, S + 1, D), x.dtype),
        compiler_params=pltpu.CompilerParams(
            dimension_semantics=("parallel", "arbitrary"),
        ),
    )
    return call(mid, x, x, tokens)
